# Initial kernel scaffold; baseline (speedup 1.0000x reference)
import jax
import jax.numpy as jnp
from jax.experimental import pallas as pl


def kernel(z, edge_index, edge_weight, edge_vec_norm, edge_attr, emb, emb2_W, emb2_b, dp1_W, dp1_b, dp2_W, dp2_b, dp3_W, dp3_b, ln_g, ln_b, mlp1_W1, mlp1_b1, mlp1_W2, mlp1_b2, lt0_W, lt1_W, lt2_W):
    raise NotImplementedError("write your pallas kernel here")


# speedup vs baseline: 42.2556x; 42.2556x over previous
import functools

import jax
import jax.numpy as jnp
import numpy as np
from jax import lax
from jax.experimental import pallas as pl
from jax.experimental.pallas import tpu as pltpu
from jax.experimental.pallas import tpu_sc as plsc

F32 = jnp.float32
I32 = jnp.int32

EP = 163840
ROWS = EP // 128
NP = 10240
EB = 2048
NB = 640
CUT_UP = 4.5


def _sc_gather_body(z_hbm, ei_hbm, out_hbm, z_v, ei_v, o_v):
    c = lax.axis_index("c")
    s = lax.axis_index("s")
    wid = s * 2 + c
    per = EP // 32
    base = wid * per
    pltpu.sync_copy(z_hbm, z_v)

    for a in range(2):
        pltpu.sync_copy(ei_hbm.at[a, pl.ds(base, per)], ei_v)

        def body(g, carry):
            idx = ei_v[pl.ds(g * 16, 16)]
            o_v[pl.ds(g * 16, 16)] = plsc.load_gather(z_v, [idx])
            return carry

        lax.fori_loop(0, per // 16, body, 0)
        pltpu.sync_copy(o_v, out_hbm.at[a, pl.ds(base, per)])


def _sc_gather(z_pad, ei_p):
    f = functools.partial(
        pl.kernel,
        out_type=jax.ShapeDtypeStruct((2, EP), I32),
        mesh=plsc.VectorSubcoreMesh(core_axis_name="c", subcore_axis_name="s"),
        scratch_types=[
            pltpu.VMEM((NP,), I32),
            pltpu.VMEM((EP // 32,), I32),
            pltpu.VMEM((EP // 32,), I32),
        ],
        compiler_params=pltpu.CompilerParams(needs_layout_passes=False),
    )(_sc_gather_body)
    return f(z_pad, ei_p)


def _payload_body(ea_ref, sc_ref, emb_ref, waT_ref, wbT_ref, e2b_ref,
                  dpcT_ref, bct_ref, out_ref):
    Ta = jnp.dot(emb_ref[...], waT_ref[...], preferred_element_type=F32)
    Tb = jnp.dot(emb_ref[...], wbT_ref[...], preferred_element_type=F32)
    w = sc_ref[0]
    vx = sc_ref[1]
    vy = sc_ref[2]
    vz = sc_ref[3]
    zi = sc_ref[4]
    zj = sc_ref[5]
    ids = lax.broadcasted_iota(I32, (EB, 128), 1)
    ohi = (zi.astype(I32)[:, None] == ids).astype(F32)
    ohj = (zj.astype(I32)[:, None] == ids).astype(F32)
    Zij = (jnp.dot(ohi, Ta, preferred_element_type=F32)
           + jnp.dot(ohj, Tb, preferred_element_type=F32)
           + e2b_ref[...][None, :])
    cut = 0.5 * (jnp.cos(w * (np.pi / CUT_UP)) + 1.0) * (w < CUT_UP).astype(F32)
    C2 = cut[:, None] * Zij
    D = jnp.dot(ea_ref[...], dpcT_ref[...], preferred_element_type=F32) \
        + bct_ref[...][None, :]
    s1 = D[:, 0:64] * C2
    s2 = D[:, 64:128] * C2
    s3 = D[:, 128:192] * C2
    t3 = (vx * vx + vy * vy + vz * vz) * (1.0 / 3.0)
    out_ref[0] = s1
    out_ref[1] = s2 * vx[:, None]
    out_ref[2] = s2 * vy[:, None]
    out_ref[3] = s2 * vz[:, None]
    out_ref[4] = s3 * (vx * vx - t3)[:, None]
    out_ref[5] = s3 * (vy * vy - t3)[:, None]
    out_ref[6] = s3 * (vz * vz - t3)[:, None]
    out_ref[7] = s3 * (vx * vy)[:, None]
    out_ref[8] = s3 * (vx * vz)[:, None]
    out_ref[9] = s3 * (vy * vz)[:, None]


def _payload(ea_p, scal, emb_p, waT, wbT, e2b, dpcT, bct):
    grid = EP // EB
    return pl.pallas_call(
        _payload_body,
        grid=(grid,),
        in_specs=[
            pl.BlockSpec((EB, 32), lambda i: (i, 0)),
            pl.BlockSpec((6, EB), lambda i: (0, i)),
            pl.BlockSpec((128, 64), lambda i: (0, 0)),
            pl.BlockSpec((64, 64), lambda i: (0, 0)),
            pl.BlockSpec((64, 64), lambda i: (0, 0)),
            pl.BlockSpec((64,), lambda i: (0,)),
            pl.BlockSpec((32, 192), lambda i: (0, 0)),
            pl.BlockSpec((192,), lambda i: (0,)),
        ],
        out_specs=pl.BlockSpec((10, EB, 64), lambda i: (0, i, 0)),
        out_shape=jax.ShapeDtypeStruct((10, EP, 64), F32),
    )(ea_p, scal, emb_p, waT, wbT, e2b, dpcT, bct)


def _sc_scatter_body(p_hbm, ei0r_hbm, g_hbm, acc, rows_v, idx_v, zrow):
    c = lax.axis_index("c")
    s = lax.axis_index("s")
    groups = ROWS // 16

    def zb(i, carry):
        zrow[i // 4, pl.ds((i % 4) * 16, 16)] = jnp.zeros((16,), F32)
        return carry

    lax.fori_loop(0, 512, zb, 0)

    for k in range(5):
        ch = c * 5 + k
        for zi_ in range(5):
            pltpu.sync_copy(zrow, acc.at[pl.ds(s * 640 + zi_ * 128, 128)])
        plsc.subcore_barrier()

        def body(g, carry):
            row = s * groups + g
            pltpu.sync_copy(ei0r_hbm.at[pl.ds(row, 1)], idx_v)
            pltpu.sync_copy(p_hbm.at[ch, pl.ds(row * 128, 128)], rows_v)
            pltpu.sync_copy(rows_v, acc.at[idx_v.at[0]], add=True)
            return carry

        lax.fori_loop(0, groups, body, 0)
        plsc.subcore_barrier()
        pltpu.sync_copy(acc.at[pl.ds(s * 640, 640)],
                        g_hbm.at[ch, pl.ds(s * 640, 640)])
        plsc.subcore_barrier()


def _sc_scatter(payload, ei0r):
    f = functools.partial(
        pl.kernel,
        out_type=jax.ShapeDtypeStruct((10, NP, 64), F32),
        mesh=plsc.VectorSubcoreMesh(core_axis_name="c", subcore_axis_name="s"),
        scratch_types=[
            pltpu.VMEM_SHARED((NP, 64), F32),
            pltpu.VMEM((128, 64), F32),
            pltpu.VMEM((1, 128), I32),
            pltpu.VMEM((128, 64), F32),
        ],
        compiler_params=pltpu.CompilerParams(needs_layout_passes=False),
    )(_sc_scatter_body)
    return f(payload, ei0r)


def _silu(x):
    return x * jax.nn.sigmoid(x)


def _epi_body(g_ref, lng_ref, lnb_ref, w1T_ref, b1_ref, w2T_ref, b2_ref,
              lt0T_ref, lt1T_ref, lt2T_ref, out_ref):
    sI = g_ref[0]
    a0 = g_ref[1]
    a1 = g_ref[2]
    a2 = g_ref[3]
    d0 = g_ref[4]
    d1 = g_ref[5]
    d2 = g_ref[6]
    o01 = g_ref[7]
    o02 = g_ref[8]
    o12 = g_ref[9]
    normf = (3.0 * sI * sI + 2.0 * (a0 * a0 + a1 * a1 + a2 * a2)
             + d0 * d0 + d1 * d1 + d2 * d2
             + 2.0 * (o01 * o01 + o02 * o02 + o12 * o12))
    mu = jnp.mean(normf, axis=-1, keepdims=True)
    var = jnp.mean((normf - mu) ** 2, axis=-1, keepdims=True)
    x = (normf - mu) * lax.rsqrt(var + 1e-5) * lng_ref[...][None, :] \
        + lnb_ref[...][None, :]
    x = _silu(jnp.dot(x, w1T_ref[...], preferred_element_type=F32)
              + b1_ref[...][None, :])
    x = _silu(jnp.dot(x, w2T_ref[...], preferred_element_type=F32)
              + b2_ref[...][None, :])
    r = lax.broadcasted_iota(I32, (192, 64), 0)
    cidx = lax.broadcasted_iota(I32, (192, 64), 1)
    n0 = jnp.dot(x, (r == 3 * cidx).astype(F32), preferred_element_type=F32)
    n1 = jnp.dot(x, (r == 3 * cidx + 1).astype(F32), preferred_element_type=F32)
    n2 = jnp.dot(x, (r == 3 * cidx + 2).astype(F32), preferred_element_type=F32)
    lt0T = lt0T_ref[...]
    lt1T = lt1T_ref[...]
    lt2T = lt2T_ref[...]
    ip = jnp.dot(sI, lt0T, preferred_element_type=F32) * n0
    a0p = jnp.dot(a0, lt1T, preferred_element_type=F32) * n1
    a1p = jnp.dot(a1, lt1T, preferred_element_type=F32) * n1
    a2p = jnp.dot(a2, lt1T, preferred_element_type=F32) * n1
    d0p = jnp.dot(d0, lt2T, preferred_element_type=F32) * n2
    d1p = jnp.dot(d1, lt2T, preferred_element_type=F32) * n2
    d2p = jnp.dot(d2, lt2T, preferred_element_type=F32) * n2
    o01p = jnp.dot(o01, lt2T, preferred_element_type=F32) * n2
    o02p = jnp.dot(o02, lt2T, preferred_element_type=F32) * n2
    o12p = jnp.dot(o12, lt2T, preferred_element_type=F32) * n2
    out_ref[0] = ip + d0p
    out_ref[1] = -a2p + o01p
    out_ref[2] = a1p + o02p
    out_ref[3] = a2p + o01p
    out_ref[4] = ip + d1p
    out_ref[5] = -a0p + o12p
    out_ref[6] = -a1p + o02p
    out_ref[7] = a0p + o12p
    out_ref[8] = ip + d2p


def _epilogue(G, ln_g, ln_b, w1T, b1, w2T, b2, lt0T, lt1T, lt2T):
    grid = NP // NB
    vec = lambda n: pl.BlockSpec((n,), lambda i: (0,))
    mat = lambda a, b: pl.BlockSpec((a, b), lambda i: (0, 0))
    return pl.pallas_call(
        _epi_body,
        grid=(grid,),
        in_specs=[
            pl.BlockSpec((10, NB, 64), lambda i: (0, i, 0)),
            vec(64), vec(64), mat(64, 128), vec(128), mat(128, 192), vec(192),
            mat(64, 64), mat(64, 64), mat(64, 64),
        ],
        out_specs=pl.BlockSpec((9, NB, 64), lambda i: (0, i, 0)),
        out_shape=jax.ShapeDtypeStruct((9, NP, 64), F32),
    )(G, ln_g, ln_b, w1T, b1, w2T, b2, lt0T, lt1T, lt2T)


def kernel(z, edge_index, edge_weight, edge_vec_norm, edge_attr, emb,
           emb2_W, emb2_b, dp1_W, dp1_b, dp2_W, dp2_b, dp3_W, dp3_b,
           ln_g, ln_b, mlp1_W1, mlp1_b1, mlp1_W2, mlp1_b2,
           lt0_W, lt1_W, lt2_W):
    E0 = edge_weight.shape[0]
    N0 = z.shape[0]
    pad = EP - E0

    ei = edge_index.astype(I32)
    ei_p = jnp.concatenate([ei, jnp.zeros((2, pad), I32)], axis=1)
    eir = ei_p.reshape(2, ROWS, 128)
    z_pad = jnp.concatenate([z.astype(I32), jnp.zeros((NP - N0,), I32)])

    zz = _sc_gather(z_pad, ei_p)
    zi = zz[0].astype(F32)
    zj = zz[1].astype(F32)

    w_p = jnp.concatenate([edge_weight, jnp.full((pad,), 10.0, F32)])
    v_p = jnp.concatenate([edge_vec_norm, jnp.zeros((pad, 3), F32)], axis=0)
    ea_p = jnp.concatenate([edge_attr, jnp.zeros((pad, 32), F32)], axis=0)
    scal = jnp.stack([w_p, v_p[:, 0], v_p[:, 1], v_p[:, 2], zi, zj])

    emb_pad = jnp.concatenate(
        [emb, jnp.zeros((128 - emb.shape[0], 64), F32)], axis=0)
    e2WT = emb2_W.T
    waT = e2WT[:64]
    wbT = e2WT[64:]
    dpcT = jnp.concatenate([dp1_W, dp2_W, dp3_W], axis=0).T
    bct = jnp.concatenate([dp1_b, dp2_b, dp3_b])

    P = _payload(ea_p, scal, emb_pad, waT, wbT, emb2_b, dpcT, bct)

    G = _sc_scatter(P, eir[0])

    out9 = _epilogue(G, ln_g, ln_b, mlp1_W1.T, mlp1_b1, mlp1_W2.T, mlp1_b2,
                     lt0_W.T, lt1_W.T, lt2_W.T)

    out = jnp.transpose(out9[:, :N0, :], (1, 2, 0)).reshape(N0, 64, 3, 3)
    return out
